# baseline (speedup 1.0000x reference)
import functools

import jax
import jax.numpy as jnp
from jax import lax
from jax.experimental import pallas as pl
from jax.experimental.pallas import tpu as pltpu
from jax.experimental.pallas import tpu_sc as plsc

VOCAB = 50760
TYPES = 7
D = 64
B, S = 4096, 200
N = B * S
EPS = 1e-12

NC, NS = 2, 16
NW = NC * NS
NPW = N // NW
CH = 128
NCHUNK = NPW // CH
NBUF = 3


_DNUMS = lax.GatherDimensionNumbers(
    offset_dims=(), collapsed_slice_dims=(0,), start_index_map=(0,))


def _permute(x, idx):
    return lax.gather(x, idx, dimension_numbers=_DNUMS, slice_sizes=(1,),
                      mode=lax.GatherScatterMode.PROMISE_IN_BOUNDS)


_mesh = plsc.VectorSubcoreMesh(core_axis_name="c", subcore_axis_name="s")


@functools.partial(
    pl.kernel,
    out_type=jax.ShapeDtypeStruct((N, D), jnp.float32),
    mesh=_mesh,
    compiler_params=pltpu.CompilerParams(use_tc_tiling_on_sc=False),
    scratch_types=[
        pltpu.VMEM((NCHUNK, CH), jnp.int32),
        pltpu.VMEM((NCHUNK, CH), jnp.int32),
        pltpu.VMEM_SHARED((TYPES, D), jnp.float32),
        pltpu.VMEM((CH, D), jnp.float32),
        pltpu.VMEM((CH, D), jnp.float32),
        pltpu.VMEM((CH, D), jnp.float32),
        pltpu.VMEM((CH, D), jnp.float32),
        pltpu.VMEM((D,), jnp.float32),
        pltpu.VMEM((D,), jnp.float32),
        pltpu.SemaphoreType.DMA,
        pltpu.SemaphoreType.DMA,
        pltpu.SemaphoreType.DMA,
        pltpu.SemaphoreType.DMA,
        pltpu.SemaphoreType.DMA,
        pltpu.SemaphoreType.DMA,
    ],
)
def _emb_ln_kernel(ids_hbm, tids_hbm, table_hbm, ttab_hbm, gamma_hbm, beta_hbm,
                   out_hbm, idx_all, tidx_all, ttab_sh, rows0, rows1,
                   outb0, outb1, g_v, b_v,
                   sg0, sg1, st0, st1, sw0, sw1):
    rows = (rows0, rows1)
    outs = (outb0, outb1)
    sg = (sg0, sg1)
    st = (st0, st1)
    sw = (sw0, sw1)

    sid = lax.axis_index("s")
    wid = sid * NC + lax.axis_index("c")
    base0 = wid * NPW
    pltpu.sync_copy(gamma_hbm, g_v)
    pltpu.sync_copy(beta_hbm, b_v)

    @pl.when(sid == 0)
    def _():
        pltpu.sync_copy(ttab_hbm, ttab_sh)

    pltpu.sync_copy(ids_hbm.at[wid], idx_all)
    pltpu.sync_copy(tids_hbm.at[wid], tidx_all)
    plsc.subcore_barrier()

    gs = [g_v[pl.ds(16 * j, 16)] for j in range(4)]
    bs = [b_v[pl.ds(16 * j, 16)] for j in range(4)]

    lanes = lax.iota(jnp.int32, 16)
    perms = [lax.reshape(lanes ^ k, (16, 1)) for k in (1, 2, 4, 8)]
    idx_lo = jnp.zeros((16, 1), jnp.int32)
    idx_hi = jnp.full((16, 1), 8, jnp.int32)
    lo_mask = lanes < 8
    c_inv_d = jnp.full((16,), 1.0 / D, jnp.float32)
    c_15 = jnp.full((16,), 1.5, jnp.float32)
    c_half = jnp.full((16,), 0.5, jnp.float32)
    c_eps = jnp.full((16,), EPS, jnp.float32)
    c_magic = jnp.full((16,), 0x5F3759DF, jnp.int32)

    def issue_g(c, b):
        pltpu.async_copy(table_hbm.at[idx_all.at[c]], rows[b], sg[b])

    def wait_g(b):
        pltpu.make_async_copy(table_hbm.at[idx_all.at[0]], rows[b], sg[b]).wait()

    def issue_tadd(c, b):
        pltpu.async_copy(ttab_sh.at[tidx_all.at[c]], rows[b], st[b], add=True)

    def wait_tadd(b):
        pltpu.make_async_copy(ttab_sh.at[tidx_all.at[0]], rows[b], st[b]).wait()

    def wait_wb(b):
        pltpu.make_async_copy(outs[b], out_hbm.at[pl.ds(0, CH)], sw[b]).wait()

    def compute(b):
        rv, ov = rows[b], outs[b]

        def pair_body(p, carry):
            i0 = p * 2
            i1 = i0 + 1
            xi = [rv[i0, pl.ds(16 * j, 16)] for j in range(4)]
            xk = [rv[i1, pl.ds(16 * j, 16)] for j in range(4)]
            si = (xi[0] + xi[1]) + (xi[2] + xi[3])
            sk = (xk[0] + xk[1]) + (xk[2] + xk[3])
            qi = (xi[0] * xi[0] + xi[1] * xi[1]) + (xi[2] * xi[2] + xi[3] * xi[3])
            qk = (xk[0] * xk[0] + xk[1] * xk[1]) + (xk[2] * xk[2] + xk[3] * xk[3])
            sm = jnp.where(lo_mask, si + _permute(si, perms[3]),
                           sk + _permute(sk, perms[3]))
            qm = jnp.where(lo_mask, qi + _permute(qi, perms[3]),
                           qk + _permute(qk, perms[3]))
            for perm in perms[:3]:
                sm = sm + _permute(sm, perm)
                qm = qm + _permute(qm, perm)
            mu = sm * c_inv_d
            var = qm * c_inv_d - mu * mu + c_eps
            hv = c_half * var
            y = lax.bitcast_convert_type(
                c_magic - (lax.bitcast_convert_type(var, jnp.int32) >> 1),
                jnp.float32)
            y = y * (c_15 - hv * y * y)
            y = y * (c_15 - hv * y * y)
            mu_i = _permute(mu, idx_lo)
            mu_k = _permute(mu, idx_hi)
            a_i = _permute(y, idx_lo)
            a_k = _permute(y, idx_hi)
            for j in range(4):
                ov[i0, pl.ds(16 * j, 16)] = (xi[j] - mu_i) * (a_i * gs[j]) + bs[j]
                ov[i1, pl.ds(16 * j, 16)] = (xk[j] - mu_k) * (a_k * gs[j]) + bs[j]
            return carry

        lax.fori_loop(0, CH // 2, pair_body, 0, unroll=1)

    def writeback(c, b):
        pltpu.async_copy(outs[b], out_hbm.at[pl.ds(base0 + c * CH, CH)], sw[b])

    def slot(c, b, do_wait_wb, pf_g, pf_t):
        wait_tadd(b)
        if do_wait_wb:
            wait_wb(b)
        compute(b)
        writeback(c, b)
        if pf_g:
            issue_g(c + 2, b)
        if pf_t:
            wait_g(1 - b)
            issue_tadd(c + 1, 1 - b)

    issue_g(0, 0)
    issue_g(1, 1)
    wait_g(0)
    issue_tadd(0, 0)

    slot(0, 0, False, True, True)
    slot(1, 1, False, True, True)

    def main_body(it, carry):
        c0 = it * 2
        slot(c0, 0, True, True, True)
        slot(c0 + 1, 1, True, True, True)
        return carry

    lax.fori_loop(1, NCHUNK // 2 - 1, main_body, 0)

    slot(NCHUNK - 2, 0, True, False, True)
    slot(NCHUNK - 1, 1, True, False, False)
    wait_wb(0)
    wait_wb(1)


def kernel(input_ids, type_ids, dpe_ids, times, input_table, type_table, gamma, beta):
    del dpe_ids, times
    ids = input_ids.reshape(NW, NCHUNK, CH)
    tids = type_ids.reshape(NW, NCHUNK, CH)
    out = _emb_ln_kernel(ids, tids, input_table, type_table, gamma, beta)
    return out.reshape(B, S, D)


# speedup vs baseline: 1.3431x; 1.3431x over previous
import functools

import jax
import jax.numpy as jnp
from jax import lax
from jax.experimental import pallas as pl
from jax.experimental.pallas import tpu as pltpu
from jax.experimental.pallas import tpu_sc as plsc

VOCAB = 50760
TYPES = 7
D = 64
DP = 128
B, S = 4096, 200
N = B * S
EPS = 1e-12

NC, NS = 2, 16
NW = NC * NS
NPW = N // NW
CH = 128
NCHUNK = NPW // CH


_DNUMS = lax.GatherDimensionNumbers(
    offset_dims=(), collapsed_slice_dims=(0,), start_index_map=(0,))


def _permute(x, idx):
    return lax.gather(x, idx, dimension_numbers=_DNUMS, slice_sizes=(1,),
                      mode=lax.GatherScatterMode.PROMISE_IN_BOUNDS)


_mesh = plsc.VectorSubcoreMesh(core_axis_name="c", subcore_axis_name="s")


@functools.partial(
    pl.kernel,
    out_type=jax.ShapeDtypeStruct((N, D), jnp.float32),
    mesh=_mesh,
    compiler_params=pltpu.CompilerParams(use_tc_tiling_on_sc=True),
    scratch_types=[
        pltpu.VMEM((NCHUNK, CH), jnp.int32),
        pltpu.VMEM((NCHUNK, CH), jnp.int32),
        pltpu.VMEM_SHARED((TYPES, DP), jnp.float32),
        pltpu.VMEM((CH, DP), jnp.float32),
        pltpu.VMEM((CH, DP), jnp.float32),
        pltpu.VMEM((CH, D), jnp.float32),
        pltpu.VMEM((CH, D), jnp.float32),
        pltpu.VMEM((D,), jnp.float32),
        pltpu.VMEM((D,), jnp.float32),
        pltpu.SemaphoreType.DMA,
        pltpu.SemaphoreType.DMA,
        pltpu.SemaphoreType.DMA,
        pltpu.SemaphoreType.DMA,
        pltpu.SemaphoreType.DMA,
        pltpu.SemaphoreType.DMA,
    ],
)
def _emb_ln_kernel(ids_hbm, tids_hbm, table_hbm, ttab_hbm, gamma_hbm, beta_hbm,
                   out_hbm, idx_all, tidx_all, ttab_sh, rows0, rows1,
                   outb0, outb1, g_v, b_v, sg0, sg1, st0, st1, sw0, sw1):
    rows = (rows0, rows1)
    outs = (outb0, outb1)
    sg = (sg0, sg1)
    st = (st0, st1)
    sw = (sw0, sw1)

    sid = lax.axis_index("s")
    wid = sid * NC + lax.axis_index("c")
    base0 = wid * NPW
    pltpu.sync_copy(gamma_hbm, g_v)
    pltpu.sync_copy(beta_hbm, b_v)

    @pl.when(sid == 0)
    def _():
        pltpu.sync_copy(ttab_hbm, ttab_sh)

    pltpu.sync_copy(ids_hbm.at[wid], idx_all)
    pltpu.sync_copy(tids_hbm.at[wid], tidx_all)
    plsc.subcore_barrier()

    gs = [g_v[pl.ds(16 * j, 16)] for j in range(4)]
    bs = [b_v[pl.ds(16 * j, 16)] for j in range(4)]

    lanes = lax.iota(jnp.int32, 16)
    perms = [lax.reshape(lanes ^ k, (16, 1)) for k in (1, 2, 4, 8)]
    idx_lo = jnp.zeros((16, 1), jnp.int32)
    idx_hi = jnp.full((16, 1), 8, jnp.int32)
    lo_mask = lanes < 8
    c_inv_d = jnp.full((16,), 1.0 / D, jnp.float32)
    c_15 = jnp.full((16,), 1.5, jnp.float32)
    c_half = jnp.full((16,), 0.5, jnp.float32)
    c_eps = jnp.full((16,), EPS, jnp.float32)
    c_magic = jnp.full((16,), 0x5F3759DF, jnp.int32)

    def issue_g(c, b):
        pltpu.async_copy(table_hbm.at[idx_all.at[c]], rows[b], sg[b])

    def wait_g(b):
        pltpu.make_async_copy(table_hbm.at[idx_all.at[0]], rows[b], sg[b]).wait()

    def issue_tadd(c, b):
        pltpu.async_copy(ttab_sh.at[tidx_all.at[c]], rows[b], st[b], add=True)

    def wait_tadd(b):
        pltpu.make_async_copy(ttab_sh.at[tidx_all.at[0]], rows[b], st[b]).wait()

    def wait_wb(b):
        pltpu.make_async_copy(outs[b], out_hbm.at[pl.ds(0, CH)], sw[b]).wait()

    def compute(b):
        rv, ov = rows[b], outs[b]

        def pair_body(p, carry):
            i0 = p * 2
            i1 = i0 + 1
            xi = [rv[i0, pl.ds(16 * j, 16)] for j in range(4)]
            xk = [rv[i1, pl.ds(16 * j, 16)] for j in range(4)]
            si = (xi[0] + xi[1]) + (xi[2] + xi[3])
            sk = (xk[0] + xk[1]) + (xk[2] + xk[3])
            qi = (xi[0] * xi[0] + xi[1] * xi[1]) + (xi[2] * xi[2] + xi[3] * xi[3])
            qk = (xk[0] * xk[0] + xk[1] * xk[1]) + (xk[2] * xk[2] + xk[3] * xk[3])
            sm = jnp.where(lo_mask, si + _permute(si, perms[3]),
                           sk + _permute(sk, perms[3]))
            qm = jnp.where(lo_mask, qi + _permute(qi, perms[3]),
                           qk + _permute(qk, perms[3]))
            for perm in perms[:3]:
                sm = sm + _permute(sm, perm)
                qm = qm + _permute(qm, perm)
            mu = sm * c_inv_d
            var = qm * c_inv_d - mu * mu + c_eps
            hv = c_half * var
            y = lax.bitcast_convert_type(
                c_magic - (lax.bitcast_convert_type(var, jnp.int32) >> 1),
                jnp.float32)
            y = y * (c_15 - hv * y * y)
            y = y * (c_15 - hv * y * y)
            mu_i = _permute(mu, idx_lo)
            mu_k = _permute(mu, idx_hi)
            a_i = _permute(y, idx_lo)
            a_k = _permute(y, idx_hi)
            for j in range(4):
                ov[i0, pl.ds(16 * j, 16)] = (xi[j] - mu_i) * (a_i * gs[j]) + bs[j]
                ov[i1, pl.ds(16 * j, 16)] = (xk[j] - mu_k) * (a_k * gs[j]) + bs[j]
            return carry

        lax.fori_loop(0, CH // 2, pair_body, 0, unroll=1)

    def writeback(c, b):
        pltpu.async_copy(outs[b], out_hbm.at[pl.ds(base0 + c * CH, CH)], sw[b])

    def slot(c, b, do_wait_wb, pf_g, pf_t):
        wait_tadd(b)
        if do_wait_wb:
            wait_wb(b)
        compute(b)
        writeback(c, b)
        if pf_g:
            issue_g(c + 2, b)
        if pf_t:
            wait_g(1 - b)
            issue_tadd(c + 1, 1 - b)

    issue_g(0, 0)
    issue_g(1, 1)
    wait_g(0)
    issue_tadd(0, 0)

    slot(0, 0, False, True, True)
    slot(1, 1, False, True, True)

    def main_body(it, carry):
        c0 = it * 2
        slot(c0, 0, True, True, True)
        slot(c0 + 1, 1, True, True, True)
        return carry

    lax.fori_loop(1, NCHUNK // 2 - 1, main_body, 0)

    slot(NCHUNK - 2, 0, True, False, True)
    slot(NCHUNK - 1, 1, True, False, False)
    wait_wb(0)
    wait_wb(1)


def kernel(input_ids, type_ids, dpe_ids, times, input_table, type_table, gamma, beta):
    del dpe_ids, times
    table_p = jnp.pad(input_table, ((0, 0), (0, DP - D)))
    ttab_p = jnp.pad(type_table, ((0, 0), (0, DP - D)))
    ids = input_ids.reshape(NW, NCHUNK, CH)
    tids = type_ids.reshape(NW, NCHUNK, CH)
    out = _emb_ln_kernel(ids, tids, table_p, ttab_p, gamma, beta)
    return out.reshape(B, S, D)
